# Initial kernel scaffold; baseline (speedup 1.0000x reference)
import jax
import jax.numpy as jnp
from jax.experimental import pallas as pl


def kernel(x_user, x_item, edge_index_ui, edge_index_iu, edge_label_index, snap, W1_ui, b1_ui, W1_iu, b1_iu, Wp1, bp1, Ws1, bs1, qs1, W2_ui, b2_ui, W2_iu, b2_iu, Wp2, bp2, Ws2, bs2, qs2, Wpost, bpost):
    raise NotImplementedError("write your pallas kernel here")


# speedup vs baseline: 6.9505x; 6.9505x over previous
import functools

import jax
import jax.numpy as jnp
from jax import lax
from jax.experimental import pallas as pl
from jax.experimental.pallas import tpu as pltpu
from jax.experimental.pallas import tpu_sc as plsc

F32 = jnp.float32
I32 = jnp.int32

NPAD = 10240
CW = 128
ROWS_PER_SUB = NPAD // 16


def _zero_rows(rows, width):
  z = jnp.zeros((16,), F32)

  def body(r, _):
    for h in range(width // 16):
      rows[r, pl.ds(h * 16, 16)] = z
    return 0

  lax.fori_loop(0, 128, body, 0)


def _zero_vec(buf, n):
  z = jnp.zeros((16,), F32)
  for k in range(n // 16):
    buf[pl.ds(k * 16, 16)] = z


def _make_scatter_kernel(h, e_pad, with_deg):
  n_per_sub = e_pad // 16
  n_chunks = n_per_sub // CW
  assert n_chunks * CW == n_per_sub

  mesh = plsc.VectorSubcoreMesh(core_axis_name="c", subcore_axis_name="s",
                                num_cores=2, num_subcores=16)
  out_type = [
      jax.ShapeDtypeStruct((NPAD, h), F32),
      jax.ShapeDtypeStruct((NPAD, h), F32),
  ]
  scratch = [
      pltpu.VMEM_SHARED((NPAD, h), F32),
      pltpu.VMEM((CW,), I32),
      pltpu.VMEM((CW,), I32),
      pltpu.VMEM((CW, h), F32),
      pltpu.VMEM((CW,), F32),
      pltpu.SemaphoreType.DMA,
  ]
  if with_deg:
    out_type += [
        jax.ShapeDtypeStruct((NPAD,), F32),
        jax.ShapeDtypeStruct((NPAD,), F32),
    ]
    scratch.insert(1, pltpu.VMEM_SHARED((NPAD,), F32))

  def body(*refs):
    if with_deg:
      (ta, tb, sa, da, sb, db, oa, ob, dega, degb,
       acc, dacc, sidx, didx, rows, fbuf, sem) = refs
    else:
      (ta, tb, sa, da, sb, db, oa, ob,
       acc, sidx, didx, rows, fbuf, sem) = refs
      dacc = dega = degb = None
    c = lax.axis_index("c")
    s = lax.axis_index("s")
    r0 = s * ROWS_PER_SUB

    _zero_rows(rows, h)
    for k in range(ROWS_PER_SUB // CW):
      pltpu.sync_copy(rows, acc.at[pl.ds(r0 + k * CW, CW)])
    if with_deg:
      _zero_vec(fbuf, CW)
      for k in range(ROWS_PER_SUB // CW):
        pltpu.sync_copy(fbuf, dacc.at[pl.ds(r0 + k * CW, CW)])
      one = jnp.ones((16,), F32)
      for k in range(CW // 16):
        fbuf[pl.ds(k * 16, 16)] = one
    plsc.subcore_barrier()

    def process(table, src, dst):
      def chunk(j, _):
        base = s * n_per_sub + j * CW
        pltpu.sync_copy(src.at[pl.ds(base, CW)], sidx)
        pltpu.sync_copy(dst.at[pl.ds(base, CW)], didx)
        pltpu.async_copy(table.at[sidx], rows, sem).wait()
        pltpu.sync_copy(rows, acc.at[didx], add=True)
        if with_deg:
          pltpu.sync_copy(fbuf, dacc.at[didx], add=True)
        return 0

      lax.fori_loop(0, n_chunks, chunk, 0)

    @pl.when(c == 0)
    def _():
      process(ta, sa, da)

    @pl.when(c == 1)
    def _():
      process(tb, sb, db)

    plsc.subcore_barrier()

    def writeout(out, deg):
      for k in range(ROWS_PER_SUB // CW):
        pltpu.sync_copy(acc.at[pl.ds(r0 + k * CW, CW)], rows)
        pltpu.sync_copy(rows, out.at[pl.ds(r0 + k * CW, CW)])
      if with_deg:
        for k in range(ROWS_PER_SUB // CW):
          pltpu.sync_copy(dacc.at[pl.ds(r0 + k * CW, CW)], fbuf)
          pltpu.sync_copy(fbuf, deg.at[pl.ds(r0 + k * CW, CW)])

    @pl.when(c == 0)
    def _():
      writeout(oa, dega)

    @pl.when(c == 1)
    def _():
      writeout(ob, degb)

  return pl.kernel(body, out_type=out_type, mesh=mesh, scratch_types=scratch,
                   compiler_params=pltpu.CompilerParams(
                       use_tc_tiling_on_sc=False))


def _make_head_kernel(b_link, h):
  n_per_w = b_link // 32
  n_chunks = n_per_w // CW
  assert n_chunks * CW == n_per_w

  mesh = plsc.VectorSubcoreMesh(core_axis_name="c", subcore_axis_name="s",
                                num_cores=2, num_subcores=16)

  def body(u2w, i2, srcl, dstl, bsum, hout,
           sidx, didx, arows, brows, hbuf, bsv, sema, semb):
    c = lax.axis_index("c")
    s = lax.axis_index("s")
    wid = s * 2 + c
    pltpu.sync_copy(bsum, bsv)
    iota = lax.iota(I32, 16)

    def chunk(j, _):
      base = wid * n_per_w + j * CW
      pltpu.sync_copy(srcl.at[pl.ds(base, CW)], sidx)
      pltpu.sync_copy(dstl.at[pl.ds(base, CW)], didx)
      ca = pltpu.async_copy(u2w.at[sidx], arows, sema)
      cb = pltpu.async_copy(i2.at[didx], brows, semb)
      ca.wait()
      cb.wait()

      def block(k, _):
        rvec = iota + k * 16
        hv = bsv[pl.ds(0, 16)]
        for cc in range(h):
          cvec = jnp.full((16,), cc, I32)
          va = plsc.load_gather(arows, [rvec, cvec])
          vb = plsc.load_gather(brows, [rvec, cvec])
          hv = hv + va * vb
        hbuf[pl.ds(k * 16, 16)] = hv
        return 0

      lax.fori_loop(0, CW // 16, block, 0)
      pltpu.sync_copy(hbuf, hout.at[pl.ds(base, CW)])
      return 0

    lax.fori_loop(0, n_chunks, chunk, 0)

  return pl.kernel(
      body,
      out_type=jax.ShapeDtypeStruct((b_link,), F32),
      mesh=mesh,
      scratch_types=[
          pltpu.VMEM((CW,), I32),
          pltpu.VMEM((CW,), I32),
          pltpu.VMEM((CW, h), F32),
          pltpu.VMEM((CW, h), F32),
          pltpu.VMEM((CW,), F32),
          pltpu.VMEM((16,), F32),
          pltpu.SemaphoreType.DMA,
          pltpu.SemaphoreType.DMA,
      ],
    compiler_params=pltpu.CompilerParams(use_tc_tiling_on_sc=False,
                                           needs_layout_passes=False),
  )


def _dot(a, b):
  return jnp.dot(a, b, preferred_element_type=F32,
                 precision=lax.Precision.HIGHEST)


def _pre1_body(xu, xi, w1ui, w1iu, wp1, yu, yi):
  yu[...] = _dot(xu[...], w1ui[...])
  wc = _dot(w1iu[...], wp1[...])
  yi[...] = _dot(xi[...], wc)


def _combine1_body(pi, pu, di, du, b1ui, b1iu, bp1v, wp1, w2ui, w2iu, wp2,
                   item1, user1, zu, zi):
  ri = 1.0 / jnp.maximum(di[...], 1.0)
  ru = 1.0 / jnp.maximum(du[...], 1.0)
  it1 = pi[...] * ri + b1ui[...]
  bc1 = _dot(b1iu[...], wp1[...]) + bp1v[...]
  us1 = pu[...] * ru + bc1
  item1[...] = it1
  user1[...] = us1
  zu[...] = _dot(us1, w2ui[...])
  zi[...] = _dot(it1, _dot(w2iu[...], wp2[...]))


def kernel(x_user, x_item, edge_index_ui, edge_index_iu, edge_label_index,
           snap, W1_ui, b1_ui, W1_iu, b1_iu, Wp1, bp1, Ws1, bs1, qs1,
           W2_ui, b2_ui, W2_iu, b2_iu, Wp2, bp2, Ws2, bs2, qs2,
           Wpost, bpost):
  n_user, d_in = x_user.shape
  n_item = x_item.shape[0]
  h1 = W1_ui.shape[1]
  h2 = W2_ui.shape[1]
  e = edge_index_ui.shape[1]
  b_link = edge_label_index.shape[1]

  e_pad = ((e + 16 * CW - 1) // (16 * CW)) * (16 * CW)
  npd = NPAD

  xu_p = jnp.pad(x_user, ((0, npd - n_user), (0, 0)))
  xi_p = jnp.pad(x_item, ((0, npd - n_item), (0, 0)))
  fill = (npd - 240) + (jnp.arange(e_pad - e, dtype=I32) % 240)
  def pad_edges(ei):
    src = jnp.concatenate([ei[0].astype(I32), fill])
    dst = jnp.concatenate([ei[1].astype(I32), fill])
    return src, dst
  src_ui, dst_ui = pad_edges(edge_index_ui)
  src_iu, dst_iu = pad_edges(edge_index_iu)
  srcl = edge_label_index[0].astype(I32)
  dstl = edge_label_index[1].astype(I32)

  grid = 8
  blk = npd // grid
  yu, yi = pl.pallas_call(
      _pre1_body,
      grid=(grid,),
      in_specs=[
          pl.BlockSpec((blk, d_in), lambda i: (i, 0)),
          pl.BlockSpec((blk, d_in), lambda i: (i, 0)),
          pl.BlockSpec((d_in, h1), lambda i: (0, 0)),
          pl.BlockSpec((d_in, h1), lambda i: (0, 0)),
          pl.BlockSpec((h1, h1), lambda i: (0, 0)),
      ],
      out_specs=[
          pl.BlockSpec((blk, h1), lambda i: (i, 0)),
          pl.BlockSpec((blk, h1), lambda i: (i, 0)),
      ],
      out_shape=[
          jax.ShapeDtypeStruct((npd, h1), F32),
          jax.ShapeDtypeStruct((npd, h1), F32),
      ],
  )(xu_p, xi_p, W1_ui, W1_iu, Wp1)

  k2 = _make_scatter_kernel(h1, e_pad, with_deg=True)
  p_item, p_user, deg_i, deg_u = k2(yu, yi, src_ui, dst_ui, src_iu, dst_iu)
  deg_i2 = deg_i.reshape(npd, 1)
  deg_u2 = deg_u.reshape(npd, 1)

  item1p, user1p, zu, zi = pl.pallas_call(
      _combine1_body,
      grid=(grid,),
      in_specs=[
          pl.BlockSpec((blk, h1), lambda i: (i, 0)),
          pl.BlockSpec((blk, h1), lambda i: (i, 0)),
          pl.BlockSpec((blk, 1), lambda i: (i, 0)),
          pl.BlockSpec((blk, 1), lambda i: (i, 0)),
          pl.BlockSpec((1, h1), lambda i: (0, 0)),
          pl.BlockSpec((1, h1), lambda i: (0, 0)),
          pl.BlockSpec((1, h1), lambda i: (0, 0)),
          pl.BlockSpec((h1, h1), lambda i: (0, 0)),
          pl.BlockSpec((h1, h2), lambda i: (0, 0)),
          pl.BlockSpec((h1, h2), lambda i: (0, 0)),
          pl.BlockSpec((h2, h2), lambda i: (0, 0)),
      ],
      out_specs=[pl.BlockSpec((blk, h1), lambda i: (i, 0))] * 2
      + [pl.BlockSpec((blk, h2), lambda i: (i, 0))] * 2,
      out_shape=[
          jax.ShapeDtypeStruct((npd, h1), F32),
          jax.ShapeDtypeStruct((npd, h1), F32),
          jax.ShapeDtypeStruct((npd, h2), F32),
          jax.ShapeDtypeStruct((npd, h2), F32),
      ],
  )(p_item, p_user, deg_i2, deg_u2, b1_ui.reshape(1, h1),
    b1_iu.reshape(1, h1), bp1.reshape(1, h1), Wp1, W2_ui, W2_iu, Wp2)

  k4 = _make_scatter_kernel(h2, e_pad, with_deg=False)
  p2_item, p2_user = k4(zu, zi, src_ui, dst_ui, src_iu, dst_iu)

  def _combine2(pi, pu, di, du, b2ui, b2iu, bp2v, wp2, wpostt,
                item2, user2, u2w):
    ri = 1.0 / jnp.maximum(di[...], 1.0)
    ru = 1.0 / jnp.maximum(du[...], 1.0)
    it2 = pi[...] * ri + b2ui[...]
    bc2 = _dot(b2iu[...], wp2[...]) + bp2v[...]
    us2 = pu[...] * ru + bc2
    wsum = jnp.sum(wpostt[...], axis=0, keepdims=True)
    item2[...] = it2
    user2[...] = us2
    u2w[...] = us2 * wsum

  item2p, user2p, u2wp = pl.pallas_call(
      _combine2,
      grid=(grid,),
      in_specs=[
          pl.BlockSpec((blk, h2), lambda i: (i, 0)),
          pl.BlockSpec((blk, h2), lambda i: (i, 0)),
          pl.BlockSpec((blk, 1), lambda i: (i, 0)),
          pl.BlockSpec((blk, 1), lambda i: (i, 0)),
          pl.BlockSpec((1, h2), lambda i: (0, 0)),
          pl.BlockSpec((1, h2), lambda i: (0, 0)),
          pl.BlockSpec((1, h2), lambda i: (0, 0)),
          pl.BlockSpec((h2, h2), lambda i: (0, 0)),
          pl.BlockSpec((2, h2), lambda i: (0, 0)),
      ],
      out_specs=[pl.BlockSpec((blk, h2), lambda i: (i, 0))] * 3,
      out_shape=[jax.ShapeDtypeStruct((npd, h2), F32)] * 3,
  )(p2_item, p2_user, deg_i2, deg_u2, b2_ui.reshape(1, h2),
    b2_iu.reshape(1, h2), bp2.reshape(1, h2), Wp2, Wpost.T)

  bsum = jnp.broadcast_to(jnp.sum(bpost), (16,)).astype(F32)
  k6 = _make_head_kernel(b_link, h2)
  h = k6(u2wp, item2p, srcl, dstl, bsum)

  return (h, user1p[:n_user], item1p[:n_item],
          user2p[:n_user], item2p[:n_item])


# speedup vs baseline: 12.3609x; 1.7784x over previous
import functools

import jax
import jax.numpy as jnp
from jax import lax
from jax.experimental import pallas as pl
from jax.experimental.pallas import tpu as pltpu
from jax.experimental.pallas import tpu_sc as plsc

F32 = jnp.float32
I32 = jnp.int32

NPAD = 10240
CW = 128
ROWS_PER_SUB = NPAD // 16


def _zero_rows(rows, width):
  z = jnp.zeros((16,), F32)

  def body(r, _):
    for h in range(width // 16):
      rows[r, pl.ds(h * 16, 16)] = z
    return 0

  lax.fori_loop(0, 128, body, 0)


def _zero_vec(buf, n):
  z = jnp.zeros((16,), F32)
  for k in range(n // 16):
    buf[pl.ds(k * 16, 16)] = z


def _make_scatter_kernel(h, e_pad, with_deg):
  n_per_sub = e_pad // 16
  cps = n_per_sub // CW
  assert cps * CW == n_per_sub and cps % 2 == 0 and cps >= 4

  mesh = plsc.VectorSubcoreMesh(core_axis_name="c", subcore_axis_name="s",
                                num_cores=2, num_subcores=16)
  out_type = [
      jax.ShapeDtypeStruct((NPAD, h), F32),
      jax.ShapeDtypeStruct((NPAD, h), F32),
  ]
  scratch = [
      pltpu.VMEM_SHARED((NPAD, h), F32),
      pltpu.VMEM((2, CW), I32),
      pltpu.VMEM((2, CW), I32),
      pltpu.VMEM((CW, h), F32),
      pltpu.VMEM((CW, h), F32),
      pltpu.VMEM((CW,), F32),
      pltpu.SemaphoreType.DMA,
      pltpu.SemaphoreType.DMA,
      pltpu.SemaphoreType.DMA,
      pltpu.SemaphoreType.DMA,
      pltpu.SemaphoreType.DMA,
  ]
  if with_deg:
    out_type += [
        jax.ShapeDtypeStruct((NPAD,), F32),
        jax.ShapeDtypeStruct((NPAD,), F32),
    ]
    scratch.insert(1, pltpu.VMEM_SHARED((NPAD,), F32))

  def body(*refs):
    if with_deg:
      (ta, tb, ea, eb, oa, ob, dega, degb,
       acc, dacc, e0, e1, r0buf, r1buf, fbuf,
       gsem0, gsem1, isem0, isem1, dsem) = refs
    else:
      (ta, tb, ea, eb, oa, ob,
       acc, e0, e1, r0buf, r1buf, fbuf,
       gsem0, gsem1, isem0, isem1, dsem) = refs
      dacc = dega = degb = None
    c = lax.axis_index("c")
    s = lax.axis_index("s")
    r0 = s * ROWS_PER_SUB

    _zero_rows(r0buf, h)
    for k in range(ROWS_PER_SUB // CW):
      pltpu.sync_copy(r0buf, acc.at[pl.ds(r0 + k * CW, CW)])
    if with_deg:
      _zero_vec(fbuf, CW)
      for k in range(ROWS_PER_SUB // CW):
        pltpu.sync_copy(fbuf, dacc.at[pl.ds(r0 + k * CW, CW)])
      one = jnp.ones((16,), F32)
      for k in range(CW // 16):
        fbuf[pl.ds(k * 16, 16)] = one
    plsc.subcore_barrier()

    def process(table, edges):
      base = s * cps
      bufs = [(e0, r0buf, gsem0, isem0), (e1, r1buf, gsem1, isem1)]

      def gather(eb_, rb_, gs_):
        return pltpu.async_copy(table.at[eb_.at[0]], rb_, gs_)

      def scatter(eb_, rb_):
        if with_deg:
          pltpu.async_copy(fbuf, dacc.at[eb_.at[1]], dsem, add=True)
        pltpu.sync_copy(rb_, acc.at[eb_.at[1]], add=True)
        if with_deg:
          pltpu.make_async_copy(fbuf, dacc.at[eb_.at[1]], dsem).wait()

      pltpu.sync_copy(edges.at[base], e0)
      gather(e0, r0buf, gsem0)
      pltpu.async_copy(edges.at[base + 1], e1, isem1)

      def pair(jj, _):
        j0 = jj * 2
        for b in range(2):
          cur = bufs[b]
          nxt = bufs[1 - b]
          j = j0 + b
          pltpu.make_async_copy(table.at[cur[0].at[0]], cur[1], cur[2]).wait()
          pltpu.make_async_copy(edges.at[base], nxt[0], nxt[3]).wait()
          gather(nxt[0], nxt[1], nxt[2])
          scatter(cur[0], cur[1])
          pltpu.async_copy(edges.at[base + j + 2], cur[0], cur[3])
        return 0

      lax.fori_loop(0, (cps - 2) // 2, pair, 0)

      pltpu.make_async_copy(table.at[e0.at[0]], r0buf, gsem0).wait()
      pltpu.make_async_copy(edges.at[base], e1, isem1).wait()
      gather(e1, r1buf, gsem1)
      scatter(e0, r0buf)
      pltpu.make_async_copy(table.at[e1.at[0]], r1buf, gsem1).wait()
      scatter(e1, r1buf)

    @pl.when(c == 0)
    def _():
      process(ta, ea)

    @pl.when(c == 1)
    def _():
      process(tb, eb)

    plsc.subcore_barrier()

    def writeout(out, deg):
      for k in range(ROWS_PER_SUB // CW):
        pltpu.sync_copy(acc.at[pl.ds(r0 + k * CW, CW)], r0buf)
        pltpu.sync_copy(r0buf, out.at[pl.ds(r0 + k * CW, CW)])
      if with_deg:
        for k in range(ROWS_PER_SUB // CW):
          pltpu.sync_copy(dacc.at[pl.ds(r0 + k * CW, CW)], fbuf)
          pltpu.sync_copy(fbuf, deg.at[pl.ds(r0 + k * CW, CW)])

    @pl.when(c == 0)
    def _():
      writeout(oa, dega)

    @pl.when(c == 1)
    def _():
      writeout(ob, degb)

  return pl.kernel(body, out_type=out_type, mesh=mesh, scratch_types=scratch,
                   compiler_params=pltpu.CompilerParams(
                       use_tc_tiling_on_sc=False))


def _make_head_kernel(b_link, h):
  n_per_w = b_link // 32
  n_chunks = n_per_w // CW
  assert n_chunks * CW == n_per_w and n_chunks % 2 == 0 and n_chunks >= 4

  mesh = plsc.VectorSubcoreMesh(core_axis_name="c", subcore_axis_name="s",
                                num_cores=2, num_subcores=16)

  def body(u2w, i2, edges, bsum, hout,
           e0, e1, ar0, ar1, br0, br1, hbuf, bsv,
           ga0, ga1, gb0, gb1, isem0, isem1):
    c = lax.axis_index("c")
    s = lax.axis_index("s")
    wid = s * 2 + c
    base = wid * n_chunks
    pltpu.sync_copy(bsum, bsv)
    iota = lax.iota(I32, 16)
    bufs = [(e0, ar0, br0, ga0, gb0, isem0), (e1, ar1, br1, ga1, gb1, isem1)]

    def gathers(bf):
      pltpu.async_copy(u2w.at[bf[0].at[0]], bf[1], bf[3])
      pltpu.async_copy(i2.at[bf[0].at[1]], bf[2], bf[4])

    def wait_gathers(bf):
      pltpu.make_async_copy(u2w.at[bf[0].at[0]], bf[1], bf[3]).wait()
      pltpu.make_async_copy(i2.at[bf[0].at[1]], bf[2], bf[4]).wait()

    def compute(bf, j):
      def block(k, _):
        rvec = iota + k * 16
        hv0 = bsv[pl.ds(0, 16)]
        hv1 = jnp.zeros((16,), F32)
        for cc in range(h // 2):
          cv0 = jnp.full((16,), 2 * cc, I32)
          cv1 = jnp.full((16,), 2 * cc + 1, I32)
          hv0 = hv0 + (plsc.load_gather(bf[1], [rvec, cv0]) *
                       plsc.load_gather(bf[2], [rvec, cv0]))
          hv1 = hv1 + (plsc.load_gather(bf[1], [rvec, cv1]) *
                       plsc.load_gather(bf[2], [rvec, cv1]))
        hbuf[pl.ds(k * 16, 16)] = hv0 + hv1
        return 0

      lax.fori_loop(0, CW // 16, block, 0)
      pltpu.sync_copy(hbuf, hout.at[pl.ds((base + j) * CW, CW)])

    pltpu.sync_copy(edges.at[base], e0)
    gathers(bufs[0])
    pltpu.async_copy(edges.at[base + 1], e1, isem1)

    def pair(jj, _):
      j0 = jj * 2
      for b in range(2):
        cur = bufs[b]
        nxt = bufs[1 - b]
        j = j0 + b
        wait_gathers(cur)
        pltpu.make_async_copy(edges.at[base], nxt[0], nxt[5]).wait()
        gathers(nxt)
        compute(cur, j)
        pltpu.async_copy(edges.at[base + j + 2], cur[0], cur[5])
      return 0

    lax.fori_loop(0, (n_chunks - 2) // 2, pair, 0)

    wait_gathers(bufs[0])
    pltpu.make_async_copy(edges.at[base], e1, isem1).wait()
    gathers(bufs[1])
    compute(bufs[0], n_chunks - 2)
    wait_gathers(bufs[1])
    compute(bufs[1], n_chunks - 1)

  return pl.kernel(
      body,
      out_type=jax.ShapeDtypeStruct((b_link,), F32),
      mesh=mesh,
      scratch_types=[
          pltpu.VMEM((2, CW), I32),
          pltpu.VMEM((2, CW), I32),
          pltpu.VMEM((CW, h), F32),
          pltpu.VMEM((CW, h), F32),
          pltpu.VMEM((CW, h), F32),
          pltpu.VMEM((CW, h), F32),
          pltpu.VMEM((CW,), F32),
          pltpu.VMEM((16,), F32),
          pltpu.SemaphoreType.DMA,
          pltpu.SemaphoreType.DMA,
          pltpu.SemaphoreType.DMA,
          pltpu.SemaphoreType.DMA,
          pltpu.SemaphoreType.DMA,
          pltpu.SemaphoreType.DMA,
      ],
    compiler_params=pltpu.CompilerParams(use_tc_tiling_on_sc=False,
                                           needs_layout_passes=False),
  )


def _dot(a, b):
  return jnp.dot(a, b, preferred_element_type=F32,
                 precision=lax.Precision.HIGHEST)


def _pre1_body(xu, xi, w1ui, w1iu, wp1, yu, yi):
  yu[...] = _dot(xu[...], w1ui[...])
  wc = _dot(w1iu[...], wp1[...])
  yi[...] = _dot(xi[...], wc)


def _combine1_body(pi, pu, di, du, b1ui, b1iu, bp1v, wp1, w2ui, w2iu, wp2,
                   item1, user1, zu, zi):
  ri = 1.0 / jnp.maximum(di[...], 1.0)
  ru = 1.0 / jnp.maximum(du[...], 1.0)
  it1 = pi[...] * ri + b1ui[...]
  bc1 = _dot(b1iu[...], wp1[...]) + bp1v[...]
  us1 = pu[...] * ru + bc1
  item1[...] = it1
  user1[...] = us1
  zu[...] = _dot(us1, w2ui[...])
  zi[...] = _dot(it1, _dot(w2iu[...], wp2[...]))


def kernel(x_user, x_item, edge_index_ui, edge_index_iu, edge_label_index,
           snap, W1_ui, b1_ui, W1_iu, b1_iu, Wp1, bp1, Ws1, bs1, qs1,
           W2_ui, b2_ui, W2_iu, b2_iu, Wp2, bp2, Ws2, bs2, qs2,
           Wpost, bpost):
  n_user, d_in = x_user.shape
  n_item = x_item.shape[0]
  h1 = W1_ui.shape[1]
  h2 = W2_ui.shape[1]
  e = edge_index_ui.shape[1]
  b_link = edge_label_index.shape[1]

  e_pad = ((e + 32 * CW - 1) // (32 * CW)) * (32 * CW)
  npd = NPAD

  xu_p = jnp.pad(x_user, ((0, npd - n_user), (0, 0)))
  xi_p = jnp.pad(x_item, ((0, npd - n_item), (0, 0)))
  fill = (npd - 240) + (jnp.arange(e_pad - e, dtype=I32) % 240)
  def pad_edges(ei):
    src = jnp.concatenate([ei[0].astype(I32), fill]).reshape(-1, 1, CW)
    dst = jnp.concatenate([ei[1].astype(I32), fill]).reshape(-1, 1, CW)
    return jnp.concatenate([src, dst], axis=1)
  eui3 = pad_edges(edge_index_ui)
  eiu3 = pad_edges(edge_index_iu)
  elab3 = jnp.concatenate(
      [edge_label_index[0].astype(I32).reshape(-1, 1, CW),
       edge_label_index[1].astype(I32).reshape(-1, 1, CW)], axis=1)

  grid = 8
  blk = npd // grid
  yu, yi = pl.pallas_call(
      _pre1_body,
      grid=(grid,),
      in_specs=[
          pl.BlockSpec((blk, d_in), lambda i: (i, 0)),
          pl.BlockSpec((blk, d_in), lambda i: (i, 0)),
          pl.BlockSpec((d_in, h1), lambda i: (0, 0)),
          pl.BlockSpec((d_in, h1), lambda i: (0, 0)),
          pl.BlockSpec((h1, h1), lambda i: (0, 0)),
      ],
      out_specs=[
          pl.BlockSpec((blk, h1), lambda i: (i, 0)),
          pl.BlockSpec((blk, h1), lambda i: (i, 0)),
      ],
      out_shape=[
          jax.ShapeDtypeStruct((npd, h1), F32),
          jax.ShapeDtypeStruct((npd, h1), F32),
      ],
  )(xu_p, xi_p, W1_ui, W1_iu, Wp1)

  k2 = _make_scatter_kernel(h1, e_pad, with_deg=True)
  p_item, p_user, deg_i, deg_u = k2(yu, yi, eui3, eiu3)
  deg_i2 = deg_i.reshape(npd, 1)
  deg_u2 = deg_u.reshape(npd, 1)

  item1p, user1p, zu, zi = pl.pallas_call(
      _combine1_body,
      grid=(grid,),
      in_specs=[
          pl.BlockSpec((blk, h1), lambda i: (i, 0)),
          pl.BlockSpec((blk, h1), lambda i: (i, 0)),
          pl.BlockSpec((blk, 1), lambda i: (i, 0)),
          pl.BlockSpec((blk, 1), lambda i: (i, 0)),
          pl.BlockSpec((1, h1), lambda i: (0, 0)),
          pl.BlockSpec((1, h1), lambda i: (0, 0)),
          pl.BlockSpec((1, h1), lambda i: (0, 0)),
          pl.BlockSpec((h1, h1), lambda i: (0, 0)),
          pl.BlockSpec((h1, h2), lambda i: (0, 0)),
          pl.BlockSpec((h1, h2), lambda i: (0, 0)),
          pl.BlockSpec((h2, h2), lambda i: (0, 0)),
      ],
      out_specs=[pl.BlockSpec((blk, h1), lambda i: (i, 0))] * 2
      + [pl.BlockSpec((blk, h2), lambda i: (i, 0))] * 2,
      out_shape=[
          jax.ShapeDtypeStruct((npd, h1), F32),
          jax.ShapeDtypeStruct((npd, h1), F32),
          jax.ShapeDtypeStruct((npd, h2), F32),
          jax.ShapeDtypeStruct((npd, h2), F32),
      ],
  )(p_item, p_user, deg_i2, deg_u2, b1_ui.reshape(1, h1),
    b1_iu.reshape(1, h1), bp1.reshape(1, h1), Wp1, W2_ui, W2_iu, Wp2)

  k4 = _make_scatter_kernel(h2, e_pad, with_deg=False)
  p2_item, p2_user = k4(zu, zi, eui3, eiu3)

  def _combine2(pi, pu, di, du, b2ui, b2iu, bp2v, wp2, wpostt,
                item2, user2, u2w):
    ri = 1.0 / jnp.maximum(di[...], 1.0)
    ru = 1.0 / jnp.maximum(du[...], 1.0)
    it2 = pi[...] * ri + b2ui[...]
    bc2 = _dot(b2iu[...], wp2[...]) + bp2v[...]
    us2 = pu[...] * ru + bc2
    wsum = jnp.sum(wpostt[...], axis=0, keepdims=True)
    item2[...] = it2
    user2[...] = us2
    u2w[...] = us2 * wsum

  item2p, user2p, u2wp = pl.pallas_call(
      _combine2,
      grid=(grid,),
      in_specs=[
          pl.BlockSpec((blk, h2), lambda i: (i, 0)),
          pl.BlockSpec((blk, h2), lambda i: (i, 0)),
          pl.BlockSpec((blk, 1), lambda i: (i, 0)),
          pl.BlockSpec((blk, 1), lambda i: (i, 0)),
          pl.BlockSpec((1, h2), lambda i: (0, 0)),
          pl.BlockSpec((1, h2), lambda i: (0, 0)),
          pl.BlockSpec((1, h2), lambda i: (0, 0)),
          pl.BlockSpec((h2, h2), lambda i: (0, 0)),
          pl.BlockSpec((2, h2), lambda i: (0, 0)),
      ],
      out_specs=[pl.BlockSpec((blk, h2), lambda i: (i, 0))] * 3,
      out_shape=[jax.ShapeDtypeStruct((npd, h2), F32)] * 3,
  )(p2_item, p2_user, deg_i2, deg_u2, b2_ui.reshape(1, h2),
    b2_iu.reshape(1, h2), bp2.reshape(1, h2), Wp2, Wpost.T)

  bsum = jnp.broadcast_to(jnp.sum(bpost), (16,)).astype(F32)
  k6 = _make_head_kernel(b_link, h2)
  h = k6(u2wp, item2p, elab3, bsum)

  return (h, user1p[:n_user], item1p[:n_item],
          user2p[:n_user], item2p[:n_item])


# speedup vs baseline: 14.5575x; 1.1777x over previous
import functools

import jax
import jax.numpy as jnp
from jax import lax
from jax.experimental import pallas as pl
from jax.experimental.pallas import tpu as pltpu
from jax.experimental.pallas import tpu_sc as plsc

F32 = jnp.float32
I32 = jnp.int32

NPAD = 10240
CW = 128
ROWS_PER_SUB = NPAD // 16


def _zero_rows(rows, width):
  z = jnp.zeros((16,), F32)

  def body(r, _):
    for h in range(width // 16):
      rows[r, pl.ds(h * 16, 16)] = z
    return 0

  lax.fori_loop(0, 128, body, 0)


def _zero_vec(buf, n):
  z = jnp.zeros((16,), F32)
  for k in range(n // 16):
    buf[pl.ds(k * 16, 16)] = z


def _make_scatter_kernel(h, e_pad, with_deg):
  n_per_sub = e_pad // 16
  cps = n_per_sub // CW
  assert cps * CW == n_per_sub and cps % 4 == 0 and cps >= 8

  mesh = plsc.VectorSubcoreMesh(core_axis_name="c", subcore_axis_name="s",
                                num_cores=2, num_subcores=16)
  out_type = [
      jax.ShapeDtypeStruct((NPAD, h), F32),
      jax.ShapeDtypeStruct((NPAD, h), F32),
  ]
  nbuf = 4
  scratch = (
      [pltpu.VMEM_SHARED((NPAD, h), F32)]
      + [pltpu.VMEM((2, CW), I32)] * nbuf
      + [pltpu.VMEM((CW, h), F32)] * nbuf
      + [pltpu.VMEM((CW,), F32)]
      + [pltpu.SemaphoreType.DMA] * (2 * nbuf + 1)
  )
  if with_deg:
    out_type += [
        jax.ShapeDtypeStruct((NPAD,), F32),
        jax.ShapeDtypeStruct((NPAD,), F32),
    ]
    scratch.insert(1, pltpu.VMEM_SHARED((NPAD,), F32))

  def body(*refs):
    if with_deg:
      (ta, tb, ea, eb, oa, ob, dega, degb, acc, dacc) = refs[:10]
      rest = refs[10:]
    else:
      (ta, tb, ea, eb, oa, ob, acc) = refs[:7]
      rest = refs[7:]
      dacc = dega = degb = None
    ebufs = rest[:4]
    rbufs = rest[4:8]
    fbuf = rest[8]
    gsems = rest[9:13]
    isems = rest[13:17]
    dsem = rest[17]
    r0buf = rbufs[0]
    c = lax.axis_index("c")
    s = lax.axis_index("s")
    r0 = s * ROWS_PER_SUB

    _zero_rows(r0buf, h)
    for k in range(ROWS_PER_SUB // CW):
      pltpu.sync_copy(r0buf, acc.at[pl.ds(r0 + k * CW, CW)])
    if with_deg:
      _zero_vec(fbuf, CW)
      for k in range(ROWS_PER_SUB // CW):
        pltpu.sync_copy(fbuf, dacc.at[pl.ds(r0 + k * CW, CW)])
      one = jnp.ones((16,), F32)
      for k in range(CW // 16):
        fbuf[pl.ds(k * 16, 16)] = one
    plsc.subcore_barrier()

    def process(table, edges):
      base = s * cps

      def gather(b):
        pltpu.async_copy(table.at[ebufs[b].at[0]], rbufs[b], gsems[b])

      def wait_gather(b):
        pltpu.make_async_copy(table.at[ebufs[b].at[0]], rbufs[b],
                              gsems[b]).wait()

      def wait_idx(b):
        pltpu.make_async_copy(edges.at[base], ebufs[b], isems[b]).wait()

      def scatter(b):
        if with_deg:
          pltpu.async_copy(fbuf, dacc.at[ebufs[b].at[1]], dsem, add=True)
        pltpu.sync_copy(rbufs[b], acc.at[ebufs[b].at[1]], add=True)
        if with_deg:
          pltpu.make_async_copy(fbuf, dacc.at[ebufs[b].at[1]], dsem).wait()

      pltpu.sync_copy(edges.at[base], ebufs[0])
      gather(0)
      pltpu.sync_copy(edges.at[base + 1], ebufs[1])
      gather(1)
      pltpu.async_copy(edges.at[base + 2], ebufs[2], isems[2])

      def quad(jj, _):
        j0 = jj * 4
        for b in range(4):
          j = j0 + b
          wait_gather(b)
          scatter(b)
          wait_idx((b + 2) % 4)
          gather((b + 2) % 4)
          pltpu.async_copy(edges.at[base + j + 3], ebufs[(b + 3) % 4],
                           isems[(b + 3) % 4])
        return 0

      lax.fori_loop(0, cps // 4 - 1, quad, 0)

      wait_gather(0); scatter(0)
      wait_idx(2); gather(2)
      pltpu.async_copy(edges.at[base + cps - 1], ebufs[3], isems[3])
      wait_gather(1); scatter(1)
      wait_idx(3); gather(3)
      wait_gather(2); scatter(2)
      wait_gather(3); scatter(3)

    @pl.when(c == 0)
    def _():
      process(ta, ea)

    @pl.when(c == 1)
    def _():
      process(tb, eb)

    plsc.subcore_barrier()

    def writeout(out, deg):
      for k in range(ROWS_PER_SUB // CW):
        pltpu.sync_copy(acc.at[pl.ds(r0 + k * CW, CW)], r0buf)
        pltpu.sync_copy(r0buf, out.at[pl.ds(r0 + k * CW, CW)])
      if with_deg:
        for k in range(ROWS_PER_SUB // CW):
          pltpu.sync_copy(dacc.at[pl.ds(r0 + k * CW, CW)], fbuf)
          pltpu.sync_copy(fbuf, deg.at[pl.ds(r0 + k * CW, CW)])

    @pl.when(c == 0)
    def _():
      writeout(oa, dega)

    @pl.when(c == 1)
    def _():
      writeout(ob, degb)

  return pl.kernel(body, out_type=out_type, mesh=mesh, scratch_types=scratch,
                   compiler_params=pltpu.CompilerParams(
                       use_tc_tiling_on_sc=False))


def _make_head_kernel(b_link, h):
  n_per_w = b_link // 32
  n_chunks = n_per_w // CW
  assert n_chunks * CW == n_per_w and n_chunks % 2 == 0 and n_chunks >= 4

  mesh = plsc.VectorSubcoreMesh(core_axis_name="c", subcore_axis_name="s",
                                num_cores=2, num_subcores=16)

  def body(u2w, i2, edges, bsum, hout,
           e0, e1, ar0, ar1, br0, br1, hbuf, bsv,
           ga0, ga1, gb0, gb1, isem0, isem1):
    c = lax.axis_index("c")
    s = lax.axis_index("s")
    wid = s * 2 + c
    base = wid * n_chunks
    pltpu.sync_copy(bsum, bsv)
    iota = lax.iota(I32, 16)
    bufs = [(e0, ar0, br0, ga0, gb0, isem0), (e1, ar1, br1, ga1, gb1, isem1)]

    def gathers(bf):
      pltpu.async_copy(u2w.at[bf[0].at[0]], bf[1], bf[3])
      pltpu.async_copy(i2.at[bf[0].at[1]], bf[2], bf[4])

    def wait_gathers(bf):
      pltpu.make_async_copy(u2w.at[bf[0].at[0]], bf[1], bf[3]).wait()
      pltpu.make_async_copy(i2.at[bf[0].at[1]], bf[2], bf[4]).wait()

    def compute(bf, j):
      def block(k, _):
        rvec = iota + k * 16
        hv0 = bsv[pl.ds(0, 16)]
        hv1 = jnp.zeros((16,), F32)
        for cc in range(h // 2):
          cv0 = jnp.full((16,), 2 * cc, I32)
          cv1 = jnp.full((16,), 2 * cc + 1, I32)
          hv0 = hv0 + (plsc.load_gather(bf[1], [rvec, cv0]) *
                       plsc.load_gather(bf[2], [rvec, cv0]))
          hv1 = hv1 + (plsc.load_gather(bf[1], [rvec, cv1]) *
                       plsc.load_gather(bf[2], [rvec, cv1]))
        hbuf[pl.ds(k * 16, 16)] = hv0 + hv1
        return 0

      lax.fori_loop(0, CW // 16, block, 0)
      pltpu.sync_copy(hbuf, hout.at[pl.ds((base + j) * CW, CW)])

    pltpu.sync_copy(edges.at[base], e0)
    gathers(bufs[0])
    pltpu.async_copy(edges.at[base + 1], e1, isem1)

    def pair(jj, _):
      j0 = jj * 2
      for b in range(2):
        cur = bufs[b]
        nxt = bufs[1 - b]
        j = j0 + b
        wait_gathers(cur)
        pltpu.make_async_copy(edges.at[base], nxt[0], nxt[5]).wait()
        gathers(nxt)
        compute(cur, j)
        pltpu.async_copy(edges.at[base + j + 2], cur[0], cur[5])
      return 0

    lax.fori_loop(0, (n_chunks - 2) // 2, pair, 0)

    wait_gathers(bufs[0])
    pltpu.make_async_copy(edges.at[base], e1, isem1).wait()
    gathers(bufs[1])
    compute(bufs[0], n_chunks - 2)
    wait_gathers(bufs[1])
    compute(bufs[1], n_chunks - 1)

  return pl.kernel(
      body,
      out_type=jax.ShapeDtypeStruct((b_link,), F32),
      mesh=mesh,
      scratch_types=[
          pltpu.VMEM((2, CW), I32),
          pltpu.VMEM((2, CW), I32),
          pltpu.VMEM((CW, h), F32),
          pltpu.VMEM((CW, h), F32),
          pltpu.VMEM((CW, h), F32),
          pltpu.VMEM((CW, h), F32),
          pltpu.VMEM((CW,), F32),
          pltpu.VMEM((16,), F32),
          pltpu.SemaphoreType.DMA,
          pltpu.SemaphoreType.DMA,
          pltpu.SemaphoreType.DMA,
          pltpu.SemaphoreType.DMA,
          pltpu.SemaphoreType.DMA,
          pltpu.SemaphoreType.DMA,
      ],
    compiler_params=pltpu.CompilerParams(use_tc_tiling_on_sc=False,
                                           needs_layout_passes=False),
  )


def _dot(a, b):
  return jnp.dot(a, b, preferred_element_type=F32,
                 precision=lax.Precision.HIGHEST)


def _pre1_body(xu, xi, w1ui, w1iu, wp1, yu, yi):
  yu[...] = _dot(xu[...], w1ui[...])
  wc = _dot(w1iu[...], wp1[...])
  yi[...] = _dot(xi[...], wc)


def _combine1_body(pi, pu, di, du, b1ui, b1iu, bp1v, wp1, w2ui, w2iu, wp2,
                   item1, user1, zu, zi):
  ri = 1.0 / jnp.maximum(di[...], 1.0)
  ru = 1.0 / jnp.maximum(du[...], 1.0)
  it1 = pi[...] * ri + b1ui[...]
  bc1 = _dot(b1iu[...], wp1[...]) + bp1v[...]
  us1 = pu[...] * ru + bc1
  item1[...] = it1
  user1[...] = us1
  zu[...] = _dot(us1, w2ui[...])
  zi[...] = _dot(it1, _dot(w2iu[...], wp2[...]))


def kernel(x_user, x_item, edge_index_ui, edge_index_iu, edge_label_index,
           snap, W1_ui, b1_ui, W1_iu, b1_iu, Wp1, bp1, Ws1, bs1, qs1,
           W2_ui, b2_ui, W2_iu, b2_iu, Wp2, bp2, Ws2, bs2, qs2,
           Wpost, bpost):
  n_user, d_in = x_user.shape
  n_item = x_item.shape[0]
  h1 = W1_ui.shape[1]
  h2 = W2_ui.shape[1]
  e = edge_index_ui.shape[1]
  b_link = edge_label_index.shape[1]

  e_pad = ((e + 64 * CW - 1) // (64 * CW)) * (64 * CW)
  npd = NPAD

  xu_p = jnp.pad(x_user, ((0, npd - n_user), (0, 0)))
  xi_p = jnp.pad(x_item, ((0, npd - n_item), (0, 0)))
  fill = (npd - 240) + (jnp.arange(e_pad - e, dtype=I32) % 240)
  def pad_edges(ei):
    src = jnp.concatenate([ei[0].astype(I32), fill]).reshape(-1, 1, CW)
    dst = jnp.concatenate([ei[1].astype(I32), fill]).reshape(-1, 1, CW)
    return jnp.concatenate([src, dst], axis=1)
  eui3 = pad_edges(edge_index_ui)
  eiu3 = pad_edges(edge_index_iu)
  elab3 = jnp.concatenate(
      [edge_label_index[0].astype(I32).reshape(-1, 1, CW),
       edge_label_index[1].astype(I32).reshape(-1, 1, CW)], axis=1)

  grid = 8
  blk = npd // grid
  yu, yi = pl.pallas_call(
      _pre1_body,
      grid=(grid,),
      in_specs=[
          pl.BlockSpec((blk, d_in), lambda i: (i, 0)),
          pl.BlockSpec((blk, d_in), lambda i: (i, 0)),
          pl.BlockSpec((d_in, h1), lambda i: (0, 0)),
          pl.BlockSpec((d_in, h1), lambda i: (0, 0)),
          pl.BlockSpec((h1, h1), lambda i: (0, 0)),
      ],
      out_specs=[
          pl.BlockSpec((blk, h1), lambda i: (i, 0)),
          pl.BlockSpec((blk, h1), lambda i: (i, 0)),
      ],
      out_shape=[
          jax.ShapeDtypeStruct((npd, h1), F32),
          jax.ShapeDtypeStruct((npd, h1), F32),
      ],
  )(xu_p, xi_p, W1_ui, W1_iu, Wp1)

  k2 = _make_scatter_kernel(h1, e_pad, with_deg=True)
  p_item, p_user, deg_i, deg_u = k2(yu, yi, eui3, eiu3)
  deg_i2 = deg_i.reshape(npd, 1)
  deg_u2 = deg_u.reshape(npd, 1)

  item1p, user1p, zu, zi = pl.pallas_call(
      _combine1_body,
      grid=(grid,),
      in_specs=[
          pl.BlockSpec((blk, h1), lambda i: (i, 0)),
          pl.BlockSpec((blk, h1), lambda i: (i, 0)),
          pl.BlockSpec((blk, 1), lambda i: (i, 0)),
          pl.BlockSpec((blk, 1), lambda i: (i, 0)),
          pl.BlockSpec((1, h1), lambda i: (0, 0)),
          pl.BlockSpec((1, h1), lambda i: (0, 0)),
          pl.BlockSpec((1, h1), lambda i: (0, 0)),
          pl.BlockSpec((h1, h1), lambda i: (0, 0)),
          pl.BlockSpec((h1, h2), lambda i: (0, 0)),
          pl.BlockSpec((h1, h2), lambda i: (0, 0)),
          pl.BlockSpec((h2, h2), lambda i: (0, 0)),
      ],
      out_specs=[pl.BlockSpec((blk, h1), lambda i: (i, 0))] * 2
      + [pl.BlockSpec((blk, h2), lambda i: (i, 0))] * 2,
      out_shape=[
          jax.ShapeDtypeStruct((npd, h1), F32),
          jax.ShapeDtypeStruct((npd, h1), F32),
          jax.ShapeDtypeStruct((npd, h2), F32),
          jax.ShapeDtypeStruct((npd, h2), F32),
      ],
  )(p_item, p_user, deg_i2, deg_u2, b1_ui.reshape(1, h1),
    b1_iu.reshape(1, h1), bp1.reshape(1, h1), Wp1, W2_ui, W2_iu, Wp2)

  k4 = _make_scatter_kernel(h2, e_pad, with_deg=False)
  p2_item, p2_user = k4(zu, zi, eui3, eiu3)

  def _combine2(pi, pu, di, du, b2ui, b2iu, bp2v, wp2, wpostt,
                item2, user2, u2w):
    ri = 1.0 / jnp.maximum(di[...], 1.0)
    ru = 1.0 / jnp.maximum(du[...], 1.0)
    it2 = pi[...] * ri + b2ui[...]
    bc2 = _dot(b2iu[...], wp2[...]) + bp2v[...]
    us2 = pu[...] * ru + bc2
    wsum = jnp.sum(wpostt[...], axis=0, keepdims=True)
    item2[...] = it2
    user2[...] = us2
    u2w[...] = us2 * wsum

  item2p, user2p, u2wp = pl.pallas_call(
      _combine2,
      grid=(grid,),
      in_specs=[
          pl.BlockSpec((blk, h2), lambda i: (i, 0)),
          pl.BlockSpec((blk, h2), lambda i: (i, 0)),
          pl.BlockSpec((blk, 1), lambda i: (i, 0)),
          pl.BlockSpec((blk, 1), lambda i: (i, 0)),
          pl.BlockSpec((1, h2), lambda i: (0, 0)),
          pl.BlockSpec((1, h2), lambda i: (0, 0)),
          pl.BlockSpec((1, h2), lambda i: (0, 0)),
          pl.BlockSpec((h2, h2), lambda i: (0, 0)),
          pl.BlockSpec((2, h2), lambda i: (0, 0)),
      ],
      out_specs=[pl.BlockSpec((blk, h2), lambda i: (i, 0))] * 3,
      out_shape=[jax.ShapeDtypeStruct((npd, h2), F32)] * 3,
  )(p2_item, p2_user, deg_i2, deg_u2, b2_ui.reshape(1, h2),
    b2_iu.reshape(1, h2), bp2.reshape(1, h2), Wp2, Wpost.T)

  bsum = jnp.broadcast_to(jnp.sum(bpost), (16,)).astype(F32)
  k6 = _make_head_kernel(b_link, h2)
  h = k6(u2wp, item2p, elab3, bsum)

  return (h, user1p[:n_user], item1p[:n_item],
          user2p[:n_user], item2p[:n_item])


# speedup vs baseline: 15.0001x; 1.0304x over previous
import functools

import jax
import jax.numpy as jnp
from jax import lax
from jax.experimental import pallas as pl
from jax.experimental.pallas import tpu as pltpu
from jax.experimental.pallas import tpu_sc as plsc

F32 = jnp.float32
I32 = jnp.int32

NPAD = 10240
CW = 128
ROWS_PER_SUB = NPAD // 16


def _zero_rows(rows, width):
  z = jnp.zeros((16,), F32)

  def body(r, _):
    for h in range(width // 16):
      rows[r, pl.ds(h * 16, 16)] = z
    return 0

  lax.fori_loop(0, 128, body, 0)


def _zero_vec(buf, n):
  z = jnp.zeros((16,), F32)
  for k in range(n // 16):
    buf[pl.ds(k * 16, 16)] = z


def _make_scatter_kernel(h, e_pad, layer):
  n_per_sub = e_pad // 16
  cps = n_per_sub // CW
  assert cps * CW == n_per_sub and cps % 4 == 0 and cps >= 8
  with_deg = layer == 1

  mesh = plsc.VectorSubcoreMesh(core_axis_name="c", subcore_axis_name="s",
                                num_cores=2, num_subcores=16)
  out_type = [
      jax.ShapeDtypeStruct((NPAD, h), F32),
      jax.ShapeDtypeStruct((NPAD, h), F32),
  ]
  nbuf = 4
  scratch = (
      [pltpu.VMEM_SHARED((NPAD, h), F32)]
      + ([pltpu.VMEM_SHARED((NPAD,), F32)] if with_deg else [])
      + [pltpu.VMEM((2, CW), I32)] * nbuf
      + [pltpu.VMEM((CW, h), F32)] * nbuf
      + [pltpu.VMEM((CW,), F32)]
      + [pltpu.VMEM((CW,), F32)]
      + [pltpu.VMEM((h,), F32)]
      + [pltpu.VMEM((h,), F32)]
      + [pltpu.SemaphoreType.DMA] * (2 * nbuf + 1)
  )
  if with_deg:
    out_type += [
        jax.ShapeDtypeStruct((NPAD,), F32),
        jax.ShapeDtypeStruct((NPAD,), F32),
    ]
  else:
    out_type += [jax.ShapeDtypeStruct((NPAD, h), F32)]

  def body(*refs):
    if with_deg:
      (ta, tb, ea, eb, biasa, biasb, oa, ob, dega, degb, acc, dacc) = refs[:12]
      rest = refs[12:]
      rdega = rdegb = wsum = u2w = None
    else:
      (ta, tb, ea, eb, biasa, biasb, rdega, rdegb, wsum,
       oa, ob, u2w, acc) = refs[:13]
      rest = refs[13:]
      dacc = dega = degb = None
    ebufs = rest[:4]
    rbufs = rest[4:8]
    fbuf = rest[8]
    dbuf = rest[9]
    bbuf = rest[10]
    wbuf = rest[11]
    gsems = rest[12:16]
    isems = rest[16:20]
    dsem = rest[20]
    r0buf = rbufs[0]
    c = lax.axis_index("c")
    s = lax.axis_index("s")
    r0 = s * ROWS_PER_SUB

    _zero_rows(r0buf, h)
    for k in range(ROWS_PER_SUB // CW):
      pltpu.sync_copy(r0buf, acc.at[pl.ds(r0 + k * CW, CW)])
    if with_deg:
      _zero_vec(fbuf, CW)
      for k in range(ROWS_PER_SUB // CW):
        pltpu.sync_copy(fbuf, dacc.at[pl.ds(r0 + k * CW, CW)])
      one = jnp.ones((16,), F32)
      for k in range(CW // 16):
        fbuf[pl.ds(k * 16, 16)] = one
    plsc.subcore_barrier()

    def process(table, edges):
      base = s * cps

      def gather(b):
        pltpu.async_copy(table.at[ebufs[b].at[0]], rbufs[b], gsems[b])

      def wait_gather(b):
        pltpu.make_async_copy(table.at[ebufs[b].at[0]], rbufs[b],
                              gsems[b]).wait()

      def wait_idx(b):
        pltpu.make_async_copy(edges.at[base], ebufs[b], isems[b]).wait()

      def scatter(b):
        if with_deg:
          pltpu.async_copy(fbuf, dacc.at[ebufs[b].at[1]], dsem, add=True)
        pltpu.sync_copy(rbufs[b], acc.at[ebufs[b].at[1]], add=True)
        if with_deg:
          pltpu.make_async_copy(fbuf, dacc.at[ebufs[b].at[1]], dsem).wait()

      pltpu.sync_copy(edges.at[base], ebufs[0])
      gather(0)
      pltpu.sync_copy(edges.at[base + 1], ebufs[1])
      gather(1)
      pltpu.async_copy(edges.at[base + 2], ebufs[2], isems[2])

      def quad(jj, _):
        j0 = jj * 4
        for b in range(4):
          j = j0 + b
          wait_gather(b)
          scatter(b)
          wait_idx((b + 2) % 4)
          gather((b + 2) % 4)
          pltpu.async_copy(edges.at[base + j + 3], ebufs[(b + 3) % 4],
                           isems[(b + 3) % 4])
        return 0

      lax.fori_loop(0, cps // 4 - 1, quad, 0)

      wait_gather(0); scatter(0)
      wait_idx(2); gather(2)
      pltpu.async_copy(edges.at[base + cps - 1], ebufs[3], isems[3])
      wait_gather(1); scatter(1)
      wait_idx(3); gather(3)
      wait_gather(2); scatter(2)
      wait_gather(3); scatter(3)

    @pl.when(c == 0)
    def _():
      process(ta, ea)

    @pl.when(c == 1)
    def _():
      process(tb, eb)

    plsc.subcore_barrier()

    def writeout(out, bias, deg_out, rdeg_in, with_u2w):
      pltpu.sync_copy(bias, bbuf)
      if with_u2w:
        pltpu.sync_copy(wsum, wbuf)
      bias_ch = [bbuf[pl.ds(cc * 16, 16)] for cc in range(h // 16)]
      w_ch = ([wbuf[pl.ds(cc * 16, 16)] for cc in range(h // 16)]
              if with_u2w else None)
      for k in range(ROWS_PER_SUB // CW):
        blk = r0 + k * CW
        pltpu.sync_copy(acc.at[pl.ds(blk, CW)], r0buf)
        if with_deg:
          pltpu.sync_copy(dacc.at[pl.ds(blk, CW)], dbuf)
          for kk in range(CW // 16):
            d = dbuf[pl.ds(kk * 16, 16)]
            dbuf[pl.ds(kk * 16, 16)] = 1.0 / jnp.maximum(d, 1.0)
          pltpu.sync_copy(dbuf, deg_out.at[pl.ds(blk, CW)])
        else:
          pltpu.sync_copy(rdeg_in.at[pl.ds(blk, CW)], dbuf)

        def rowgrp(g, _):
          rv16 = dbuf[pl.ds(g * 16, 16)]
          for i in range(16):
            r = g * 16 + i
            rv = rv16[i]
            for cc in range(h // 16):
              x = r0buf[r, pl.ds(cc * 16, 16)]
              y = x * rv + bias_ch[cc]
              r0buf[r, pl.ds(cc * 16, 16)] = y
              if with_u2w:
                rbufs[1][r, pl.ds(cc * 16, 16)] = y * w_ch[cc]
          return 0

        lax.fori_loop(0, CW // 16, rowgrp, 0)
        pltpu.sync_copy(r0buf, out.at[pl.ds(blk, CW)])
        if with_u2w:
          pltpu.sync_copy(rbufs[1], u2w.at[pl.ds(blk, CW)])

    @pl.when(c == 0)
    def _():
      writeout(oa, biasa, dega, rdega, False)

    @pl.when(c == 1)
    def _():
      writeout(ob, biasb, degb, rdegb, not with_deg)

  return pl.kernel(body, out_type=out_type, mesh=mesh, scratch_types=scratch,
                   compiler_params=pltpu.CompilerParams(
                       use_tc_tiling_on_sc=False))


def _make_head_kernel(b_link, h):
  n_per_w = b_link // 32
  n_chunks = n_per_w // CW
  assert n_chunks * CW == n_per_w and n_chunks % 2 == 0 and n_chunks >= 4

  mesh = plsc.VectorSubcoreMesh(core_axis_name="c", subcore_axis_name="s",
                                num_cores=2, num_subcores=16)

  def body(u2w, i2, edges, bsum, hout,
           e0, e1, ar0, ar1, br0, br1, hbuf, bsv,
           ga0, ga1, gb0, gb1, isem0, isem1):
    c = lax.axis_index("c")
    s = lax.axis_index("s")
    wid = s * 2 + c
    base = wid * n_chunks
    pltpu.sync_copy(bsum, bsv)
    iota = lax.iota(I32, 16)
    bufs = [(e0, ar0, br0, ga0, gb0, isem0), (e1, ar1, br1, ga1, gb1, isem1)]

    def gathers(bf):
      pltpu.async_copy(u2w.at[bf[0].at[0]], bf[1], bf[3])
      pltpu.async_copy(i2.at[bf[0].at[1]], bf[2], bf[4])

    def wait_gathers(bf):
      pltpu.make_async_copy(u2w.at[bf[0].at[0]], bf[1], bf[3]).wait()
      pltpu.make_async_copy(i2.at[bf[0].at[1]], bf[2], bf[4]).wait()

    def compute(bf, j):
      def block(k, _):
        rvec = iota + k * 16
        hv0 = bsv[pl.ds(0, 16)]
        hv1 = jnp.zeros((16,), F32)
        for cc in range(h // 2):
          cv0 = jnp.full((16,), 2 * cc, I32)
          cv1 = jnp.full((16,), 2 * cc + 1, I32)
          hv0 = hv0 + (plsc.load_gather(bf[1], [rvec, cv0]) *
                       plsc.load_gather(bf[2], [rvec, cv0]))
          hv1 = hv1 + (plsc.load_gather(bf[1], [rvec, cv1]) *
                       plsc.load_gather(bf[2], [rvec, cv1]))
        hbuf[pl.ds(k * 16, 16)] = hv0 + hv1
        return 0

      lax.fori_loop(0, CW // 16, block, 0)
      pltpu.sync_copy(hbuf, hout.at[pl.ds((base + j) * CW, CW)])

    pltpu.sync_copy(edges.at[base], e0)
    gathers(bufs[0])
    pltpu.async_copy(edges.at[base + 1], e1, isem1)

    def pair(jj, _):
      j0 = jj * 2
      for b in range(2):
        cur = bufs[b]
        nxt = bufs[1 - b]
        j = j0 + b
        wait_gathers(cur)
        pltpu.make_async_copy(edges.at[base], nxt[0], nxt[5]).wait()
        gathers(nxt)
        compute(cur, j)
        pltpu.async_copy(edges.at[base + j + 2], cur[0], cur[5])
      return 0

    lax.fori_loop(0, (n_chunks - 2) // 2, pair, 0)

    wait_gathers(bufs[0])
    pltpu.make_async_copy(edges.at[base], e1, isem1).wait()
    gathers(bufs[1])
    compute(bufs[0], n_chunks - 2)
    wait_gathers(bufs[1])
    compute(bufs[1], n_chunks - 1)

  return pl.kernel(
      body,
      out_type=jax.ShapeDtypeStruct((b_link,), F32),
      mesh=mesh,
      scratch_types=[
          pltpu.VMEM((2, CW), I32),
          pltpu.VMEM((2, CW), I32),
          pltpu.VMEM((CW, h), F32),
          pltpu.VMEM((CW, h), F32),
          pltpu.VMEM((CW, h), F32),
          pltpu.VMEM((CW, h), F32),
          pltpu.VMEM((CW,), F32),
          pltpu.VMEM((16,), F32),
          pltpu.SemaphoreType.DMA,
          pltpu.SemaphoreType.DMA,
          pltpu.SemaphoreType.DMA,
          pltpu.SemaphoreType.DMA,
          pltpu.SemaphoreType.DMA,
          pltpu.SemaphoreType.DMA,
      ],
    compiler_params=pltpu.CompilerParams(use_tc_tiling_on_sc=False,
                                           needs_layout_passes=False),
  )


def _dot(a, b):
  return jnp.dot(a, b, preferred_element_type=F32,
                 precision=lax.Precision.HIGHEST)


def _pre1_body(xu, xi, w1ui, w1iu, wp1, b1iu, bp1v, yu, yi, bc1):
  yu[...] = _dot(xu[...], w1ui[...])
  wc = _dot(w1iu[...], wp1[...])
  yi[...] = _dot(xi[...], wc)
  bc1[...] = _dot(b1iu[...], wp1[...]) + bp1v[...]


def _pre2_body(user1, item1, w2ui, w2iu, wp2, b2iu, bp2v, wpostt,
               zu, zi, bc2, wsum):
  zu[...] = _dot(user1[...], w2ui[...])
  zi[...] = _dot(item1[...], _dot(w2iu[...], wp2[...]))
  bc2[...] = _dot(b2iu[...], wp2[...]) + bp2v[...]
  wsum[...] = jnp.sum(wpostt[...], axis=0, keepdims=True)


def kernel(x_user, x_item, edge_index_ui, edge_index_iu, edge_label_index,
           snap, W1_ui, b1_ui, W1_iu, b1_iu, Wp1, bp1, Ws1, bs1, qs1,
           W2_ui, b2_ui, W2_iu, b2_iu, Wp2, bp2, Ws2, bs2, qs2,
           Wpost, bpost):
  n_user, d_in = x_user.shape
  n_item = x_item.shape[0]
  h1 = W1_ui.shape[1]
  h2 = W2_ui.shape[1]
  e = edge_index_ui.shape[1]
  b_link = edge_label_index.shape[1]

  e_pad = ((e + 64 * CW - 1) // (64 * CW)) * (64 * CW)
  npd = NPAD

  xu_p = jnp.pad(x_user, ((0, npd - n_user), (0, 0)))
  xi_p = jnp.pad(x_item, ((0, npd - n_item), (0, 0)))
  fill = (npd - 240) + (jnp.arange(e_pad - e, dtype=I32) % 240)
  def pad_edges(ei):
    src = jnp.concatenate([ei[0].astype(I32), fill]).reshape(-1, 1, CW)
    dst = jnp.concatenate([ei[1].astype(I32), fill]).reshape(-1, 1, CW)
    return jnp.concatenate([src, dst], axis=1)
  eui3 = pad_edges(edge_index_ui)
  eiu3 = pad_edges(edge_index_iu)
  elab3 = jnp.concatenate(
      [edge_label_index[0].astype(I32).reshape(-1, 1, CW),
       edge_label_index[1].astype(I32).reshape(-1, 1, CW)], axis=1)

  grid = 8
  blk = npd // grid
  yu, yi, bc1 = pl.pallas_call(
      _pre1_body,
      grid=(grid,),
      in_specs=[
          pl.BlockSpec((blk, d_in), lambda i: (i, 0)),
          pl.BlockSpec((blk, d_in), lambda i: (i, 0)),
          pl.BlockSpec((d_in, h1), lambda i: (0, 0)),
          pl.BlockSpec((d_in, h1), lambda i: (0, 0)),
          pl.BlockSpec((h1, h1), lambda i: (0, 0)),
          pl.BlockSpec((1, h1), lambda i: (0, 0)),
          pl.BlockSpec((1, h1), lambda i: (0, 0)),
      ],
      out_specs=[
          pl.BlockSpec((blk, h1), lambda i: (i, 0)),
          pl.BlockSpec((blk, h1), lambda i: (i, 0)),
          pl.BlockSpec((1, h1), lambda i: (0, 0)),
      ],
      out_shape=[
          jax.ShapeDtypeStruct((npd, h1), F32),
          jax.ShapeDtypeStruct((npd, h1), F32),
          jax.ShapeDtypeStruct((1, h1), F32),
      ],
  )(xu_p, xi_p, W1_ui, W1_iu, Wp1, b1_iu.reshape(1, h1), bp1.reshape(1, h1))

  k2 = _make_scatter_kernel(h1, e_pad, layer=1)
  item1p, user1p, rdeg_i, rdeg_u = k2(yu, yi, eui3, eiu3,
                                      b1_ui, bc1.reshape(h1))

  zu, zi, bc2, wsum = pl.pallas_call(
      _pre2_body,
      grid=(grid,),
      in_specs=[
          pl.BlockSpec((blk, h1), lambda i: (i, 0)),
          pl.BlockSpec((blk, h1), lambda i: (i, 0)),
          pl.BlockSpec((h1, h2), lambda i: (0, 0)),
          pl.BlockSpec((h1, h2), lambda i: (0, 0)),
          pl.BlockSpec((h2, h2), lambda i: (0, 0)),
          pl.BlockSpec((1, h2), lambda i: (0, 0)),
          pl.BlockSpec((1, h2), lambda i: (0, 0)),
          pl.BlockSpec((2, h2), lambda i: (0, 0)),
      ],
      out_specs=[pl.BlockSpec((blk, h2), lambda i: (i, 0))] * 2
      + [pl.BlockSpec((1, h2), lambda i: (0, 0))] * 2,
      out_shape=[
          jax.ShapeDtypeStruct((npd, h2), F32),
          jax.ShapeDtypeStruct((npd, h2), F32),
          jax.ShapeDtypeStruct((1, h2), F32),
          jax.ShapeDtypeStruct((1, h2), F32),
      ],
  )(user1p, item1p, W2_ui, W2_iu, Wp2, b2_iu.reshape(1, h2),
    bp2.reshape(1, h2), Wpost.T)

  k4 = _make_scatter_kernel(h2, e_pad, layer=2)
  item2p, user2p, u2wp = k4(zu, zi, eui3, eiu3, b2_ui, bc2.reshape(h2),
                            rdeg_i, rdeg_u, wsum.reshape(h2))

  bsum = jnp.broadcast_to(jnp.sum(bpost), (16,)).astype(F32)
  k6 = _make_head_kernel(b_link, h2)
  h = k6(u2wp, item2p, elab3, bsum)

  return (h, user1p[:n_user], item1p[:n_item],
          user2p[:n_user], item2p[:n_item])
